# baseline (speedup 1.0000x reference)
import functools
import numpy as np
import jax
import jax.numpy as jnp
from jax import lax
from jax.experimental import pallas as pl
from jax.experimental.pallas import tpu as pltpu
from jax.experimental.pallas import tpu_sc as plsc

_N = 10000
_E = 320000
_HEADS = 4
_NTGT = 256

_NC = 2
_NS = 16
_NTILES = _NC * _NS
_CH = 80


def _pad128(w):
    return (w + 127) // 128 * 128


def _sc_mesh():
    return plsc.VectorSubcoreMesh(
        core_axis_name="c", subcore_axis_name="s",
        num_cores=_NC, num_subcores=_NS)


def _make_proj(din, dout):
    c = dout // _HEADS
    f = dout // 2
    wq = _pad128(dout + 16)
    wk = _pad128(dout)
    wv = _pad128(f)
    bm = 1000
    kcat = 4 * dout

    def body(x_ref, wcat_ref, bcat_ref, wet_ref, o_qp, o_k, o_va, o_vb, o_s):
        p = (jnp.dot(x_ref[...], wcat_ref[...],
                     preferred_element_type=jnp.float32) + bcat_ref[...])
        q = p[:, :dout]
        pt = jnp.dot(q, wet_ref[...], preferred_element_type=jnp.float32)
        pcols = [q, pt]
        if wq > dout + 16:
            pcols.append(jnp.zeros((bm, wq - dout - 16), jnp.float32))
        o_qp[...] = jnp.concatenate(pcols, axis=1)
        kcols = [p[:, dout:2 * dout]]
        if wk > dout:
            kcols.append(jnp.zeros((bm, wk - dout), jnp.float32))
        o_k[...] = jnp.concatenate(kcols, axis=1) if len(kcols) > 1 else kcols[0]
        if dout == 64:
            o_va[...] = jnp.concatenate(
                [p[:, 2 * dout:3 * dout],
                 jnp.zeros((bm, wv - dout), jnp.float32)], axis=1)
            o_vb[...] = jnp.zeros((bm, wv), jnp.float32)
        else:
            vpad = [jnp.zeros((bm, wv - f), jnp.float32)] if wv > f else []
            va = [p[:, 2 * dout:2 * dout + f]] + vpad
            vb = [p[:, 2 * dout + f:3 * dout]] + vpad
            o_va[...] = jnp.concatenate(va, axis=1) if len(va) > 1 else va[0]
            o_vb[...] = jnp.concatenate(vb, axis=1) if len(vb) > 1 else vb[0]
        o_s[...] = p[:, 3 * dout:]

    def run(x, Wcat, bcat, WeT2):
        return pl.pallas_call(
            body,
            grid=(_N // bm,),
            in_specs=[
                pl.BlockSpec((bm, din), lambda i: (i, 0)),
                pl.BlockSpec((din, kcat), lambda i: (0, 0)),
                pl.BlockSpec((1, kcat), lambda i: (0, 0)),
                pl.BlockSpec((dout, 16), lambda i: (0, 0)),
            ],
            out_specs=[
                pl.BlockSpec((bm, wq), lambda i: (i, 0)),
                pl.BlockSpec((bm, wk), lambda i: (i, 0)),
                pl.BlockSpec((bm, wv), lambda i: (i, 0)),
                pl.BlockSpec((bm, wv), lambda i: (i, 0)),
                pl.BlockSpec((bm, dout), lambda i: (i, 0)),
            ],
            out_shape=[
                jax.ShapeDtypeStruct((_N, wq), jnp.float32),
                jax.ShapeDtypeStruct((_N, wk), jnp.float32),
                jax.ShapeDtypeStruct((_N, wv), jnp.float32),
                jax.ShapeDtypeStruct((_N, wv), jnp.float32),
                jax.ShapeDtypeStruct((_N, dout), jnp.float32),
            ],
        )(x, Wcat, bcat.reshape(1, kcat), WeT2)

    return run


def _make_pass_a(dout, ch):
    c = dout // _HEADS
    nb = c // 16
    ept = _E // _NTILES
    nch = ept // ch
    wq = _pad128(dout + 16)
    wk = _pad128(dout)
    inv = float(1.0 / np.sqrt(c))

    @functools.partial(
        pl.kernel,
        out_type=jax.ShapeDtypeStruct((_E * _HEADS,), jnp.float32),
        mesh=_sc_mesh(),
        scratch_types=[
            [pltpu.VMEM((ch,), jnp.int32)] * 2,
            [pltpu.VMEM((ch,), jnp.int32)] * 2,
            [pltpu.VMEM((ch, wq), jnp.float32)] * 2,
            [pltpu.VMEM((ch, wk), jnp.float32)] * 2,
            [pltpu.VMEM((ch * 4,), jnp.float32)] * 2,
            [pltpu.VMEM((ch * 4,), jnp.float32)] * 2,
            [pltpu.SemaphoreType.DMA] * 2,
            [pltpu.SemaphoreType.DMA] * 2,
        ],
    )
    def pass_a(dsti, srci, qp, kt, ea, aexp_out,
               idxd, idxs, qb, kb, eb, ab, sg, so):
        wid = lax.axis_index("s") * _NC + lax.axis_index("c")
        base0 = wid * ept
        lanes = lax.broadcasted_iota(jnp.int32, (16,), 0)
        perms = [lanes ^ kk for kk in (8, 4, 2, 1)]

        def issue(ci, p):
            base = base0 + ci * ch
            pltpu.sync_copy(dsti.at[pl.ds(base, ch)], idxd[p])
            pltpu.sync_copy(srci.at[pl.ds(base, ch)], idxs[p])
            pltpu.async_copy(qp.at[idxd[p]], qb[p], sg[p])
            pltpu.async_copy(kt.at[idxs[p]], kb[p], sg[p])
            pltpu.async_copy(ea.at[pl.ds(base * 4, ch * 4)], eb[p], sg[p])

        def wait_in(p):
            pltpu.make_async_copy(qp.at[idxd[p]], qb[p], sg[p]).wait()
            pltpu.make_async_copy(kt.at[idxs[p]], kb[p], sg[p]).wait()
            pltpu.make_async_copy(ea.at[pl.ds(0, ch * 4)], eb[p], sg[p]).wait()

        def compute(ci, p):
            def group(g, cc):
                eav = eb[p][pl.ds(g * 16, 16)]
                vals = jnp.zeros((16,), jnp.float32)
                for u in range(4):
                    e = g * 4 + u
                    pv = qb[p][e, pl.ds(dout, 16)]
                    for h in range(_HEADS):
                        acc = (qb[p][e, pl.ds(h * c, 16)]
                               * kb[p][e, pl.ds(h * c, 16)])
                        for b in range(1, nb):
                            acc = acc + (qb[p][e, pl.ds(h * c + b * 16, 16)]
                                         * kb[p][e, pl.ds(h * c + b * 16, 16)])
                        for pp in perms:
                            acc = acc + jnp.take(acc, pp)
                        t = (eav[4 * u] * pv[4 * h]
                             + eav[4 * u + 1] * pv[4 * h + 1]
                             + eav[4 * u + 2] * pv[4 * h + 2]
                             + eav[4 * u + 3] * pv[4 * h + 3])
                        vals = jnp.where(lanes == (4 * u + h),
                                         (acc + t) * inv, vals)
                ab[p][pl.ds(g * 16, 16)] = jnp.exp(vals)
                return cc

            lax.fori_loop(0, ch // 4, group, 0)
            base = base0 + ci * ch
            pltpu.async_copy(ab[p], aexp_out.at[pl.ds(base * 4, ch * 4)],
                             so[p])

        def wait_out(p):
            pltpu.make_async_copy(ab[p], aexp_out.at[pl.ds(0, ch * 4)],
                                  so[p]).wait()

        issue(0, 0)

        def pair(cj, carry):
            for p in (0, 1):
                ci = 2 * cj + p

                @pl.when(ci + 1 < nch)
                def _():
                    issue(ci + 1, p ^ 1)

                @pl.when(ci < nch)
                def _():
                    wait_in(p)

                    @pl.when(ci >= 2)
                    def _():
                        wait_out(p)

                    compute(ci, p)

            return carry

        lax.fori_loop(0, (nch + 1) // 2, pair, 0)
        wait_out(0)
        wait_out(1)

    return pass_a


def _make_pass_b(dout, ch):
    c = dout // _HEADS
    f = dout // 2
    wv = _pad128(f)
    epsc = _E // _NS
    nch = epsc // ch
    nfull = _N // 128

    @functools.partial(
        pl.kernel,
        out_type=(
            jax.ShapeDtypeStruct((_N, wv), jnp.float32),
            jax.ShapeDtypeStruct((_N, wv), jnp.float32),
        ),
        mesh=_sc_mesh(),
        scratch_types=[
            [pltpu.VMEM((ch,), jnp.int32)] * 2,
            [pltpu.VMEM((ch,), jnp.int32)] * 2,
            [pltpu.VMEM((ch, wv), jnp.float32)] * 2,
            [pltpu.VMEM((ch * 4,), jnp.float32)] * 2,
            pltpu.VMEM((128, wv), jnp.float32),
            pltpu.VMEM_SHARED((_N, wv), jnp.float32),
            [pltpu.SemaphoreType.DMA] * 2,
            [pltpu.SemaphoreType.DMA] * 2,
        ],
    )
    def pass_b(dsti, srci, va, vb, aexp, outa, outb,
               idxd, idxs, vbuf, axbuf, zb, tabv, sg, sc):
        cid = lax.axis_index("c")
        sid = lax.axis_index("s")
        zv = jnp.zeros((16,), jnp.float32)

        def z1(r, cc):
            for b in range(wv // 16):
                zb[r, pl.ds(b * 16, 16)] = zv
            return cc

        lax.fori_loop(0, 128, z1, 0)

        def zc(j, cc):
            blk = sid + _NS * j

            @pl.when(blk < nfull)
            def _():
                pltpu.sync_copy(zb, tabv.at[pl.ds(blk * 128, 128)])

            return cc

        lax.fori_loop(0, nfull // _NS + 1, zc, 0)

        @pl.when(sid == 0)
        def _ztail():
            pltpu.sync_copy(zb.at[pl.ds(0, _N - nfull * 128)],
                            tabv.at[pl.ds(nfull * 128, _N - nfull * 128)])

        plsc.subcore_barrier()

        def wait_sc(p):
            pltpu.make_async_copy(vbuf[p], tabv.at[idxd[p]], sc[p]).wait()

        def issue(ci, p):
            @pl.when(ci >= 2)
            def _():
                wait_sc(p)

            base = sid * epsc + ci * ch
            pltpu.sync_copy(dsti.at[pl.ds(base, ch)], idxd[p])
            pltpu.sync_copy(srci.at[pl.ds(base, ch)], idxs[p])
            pltpu.async_copy(aexp.at[pl.ds(base * 4, ch * 4)], axbuf[p],
                             sg[p])

            @pl.when(cid == 0)
            def _():
                pltpu.async_copy(va.at[idxs[p]], vbuf[p], sg[p])

            @pl.when(cid == 1)
            def _():
                pltpu.async_copy(vb.at[idxs[p]], vbuf[p], sg[p])

        def compute(ci, p):
            pltpu.make_async_copy(va.at[idxs[p]], vbuf[p], sg[p]).wait()
            pltpu.make_async_copy(
                aexp.at[pl.ds(0, ch * 4)], axbuf[p], sg[p]).wait()

            def scale(coff):
                def group(g, cc):
                    axv = axbuf[p][pl.ds(g * 16, 16)]
                    for u in range(4):
                        e = g * 4 + u
                        for b in range(f // 16):
                            h = (coff + b * 16) // c
                            vbuf[p][e, pl.ds(b * 16, 16)] = (
                                vbuf[p][e, pl.ds(b * 16, 16)]
                                * axv[4 * u + h])
                    return cc

                lax.fori_loop(0, ch // 4, group, 0)

            @pl.when(cid == 0)
            def _():
                scale(0)

            @pl.when(cid == 1)
            def _():
                scale(f)

            pltpu.async_copy(vbuf[p], tabv.at[idxd[p]], sc[p], add=True)

        issue(0, 0)

        def pair(cj, carry):
            for p in (0, 1):
                ci = 2 * cj + p

                @pl.when(ci + 1 < nch)
                def _():
                    issue(ci + 1, p ^ 1)

                @pl.when(ci < nch)
                def _():
                    compute(ci, p)

            return carry

        lax.fori_loop(0, (nch + 1) // 2, pair, 0)
        wait_sc(0)
        wait_sc(1)
        plsc.subcore_barrier()

        def dump(j, cc):
            blk = sid + _NS * j

            @pl.when(blk < nfull)
            def _():
                @pl.when(cid == 0)
                def _():
                    pltpu.sync_copy(tabv.at[pl.ds(blk * 128, 128)],
                                    outa.at[pl.ds(blk * 128, 128)])

                @pl.when(cid == 1)
                def _():
                    pltpu.sync_copy(tabv.at[pl.ds(blk * 128, 128)],
                                    outb.at[pl.ds(blk * 128, 128)])

            return cc

        lax.fori_loop(0, nfull // _NS + 1, dump, 0)
        tail = _N - nfull * 128

        @pl.when(sid == 1)
        def _dtail():
            @pl.when(cid == 0)
            def _():
                pltpu.sync_copy(tabv.at[pl.ds(nfull * 128, tail)],
                                outa.at[pl.ds(nfull * 128, tail)])

            @pl.when(cid == 1)
            def _():
                pltpu.sync_copy(tabv.at[pl.ds(nfull * 128, tail)],
                                outb.at[pl.ds(nfull * 128, tail)])

    return pass_b


def _make_pass_b0(ch):
    dout = 64
    ept = _E // _NTILES
    nch = ept // ch
    nfull = _N // 128

    @functools.partial(
        pl.kernel,
        out_type=(
            jax.ShapeDtypeStruct((_N, 128), jnp.float32),
            jax.ShapeDtypeStruct((_N, 128), jnp.float32),
        ),
        mesh=_sc_mesh(),
        scratch_types=[
            [pltpu.VMEM((ch,), jnp.int32)] * 2,
            [pltpu.VMEM((ch,), jnp.int32)] * 2,
            [pltpu.VMEM((ch, 128), jnp.float32)] * 2,
            [pltpu.VMEM((ch * 4,), jnp.float32)] * 2,
            [pltpu.VMEM((ch * 4,), jnp.float32)] * 2,
            pltpu.VMEM((128, 128), jnp.float32),
            pltpu.VMEM_SHARED((_N, 128), jnp.float32),
            [pltpu.SemaphoreType.DMA] * 2,
            [pltpu.SemaphoreType.DMA] * 2,
        ],
    )
    def pass_b0(dsti, srci, vt, aexp, ea, outa, outb,
                idxd, idxs, vbuf, axbuf, eabuf, zb, tabv, sg, sc):
        cid = lax.axis_index("c")
        sid = lax.axis_index("s")
        zv = jnp.zeros((16,), jnp.float32)

        def z1(r, cc):
            for b in range(8):
                zb[r, pl.ds(b * 16, 16)] = zv
            return cc

        lax.fori_loop(0, 128, z1, 0)

        def zc(j, cc):
            blk = sid + _NS * j

            @pl.when(blk < nfull)
            def _():
                pltpu.sync_copy(zb, tabv.at[pl.ds(blk * 128, 128)])

            return cc

        lax.fori_loop(0, nfull // _NS + 1, zc, 0)

        @pl.when(sid == 0)
        def _ztail():
            pltpu.sync_copy(zb.at[pl.ds(0, _N - nfull * 128)],
                            tabv.at[pl.ds(nfull * 128, _N - nfull * 128)])

        plsc.subcore_barrier()

        lanes = lax.broadcasted_iota(jnp.int32, (16,), 0)
        zvec = jnp.zeros((16,), jnp.float32)

        def wait_sc(p):
            pltpu.make_async_copy(vbuf[p], tabv.at[idxd[p]], sc[p]).wait()

        def issue(ci, p):
            @pl.when(ci >= 2)
            def _():
                wait_sc(p)

            base = (cid * _NS + sid) * ept + ci * ch
            pltpu.sync_copy(dsti.at[pl.ds(base, ch)], idxd[p])
            pltpu.sync_copy(srci.at[pl.ds(base, ch)], idxs[p])
            pltpu.async_copy(aexp.at[pl.ds(base * 4, ch * 4)], axbuf[p],
                             sg[p])
            pltpu.async_copy(ea.at[pl.ds(base * 4, ch * 4)], eabuf[p], sg[p])
            pltpu.async_copy(vt.at[idxs[p]], vbuf[p], sg[p])

        def compute(ci, p):
            pltpu.make_async_copy(vt.at[idxs[p]], vbuf[p], sg[p]).wait()
            pltpu.make_async_copy(
                aexp.at[pl.ds(0, ch * 4)], axbuf[p], sg[p]).wait()
            pltpu.make_async_copy(
                ea.at[pl.ds(0, ch * 4)], eabuf[p], sg[p]).wait()

            def group(g, cc):
                axv = axbuf[p][pl.ds(g * 16, 16)]
                eav = eabuf[p][pl.ds(g * 16, 16)]
                for u in range(4):
                    e = g * 4 + u
                    for b in range(4):
                        vbuf[p][e, pl.ds(b * 16, 16)] = (
                            vbuf[p][e, pl.ds(b * 16, 16)] * axv[4 * u + b])
                    ia = (lanes >> 2) + 4 * u
                    ie = (lanes & 3) + 4 * u
                    it = jnp.minimum(lanes, 3) + 4 * u
                    vbuf[p][e, pl.ds(64, 16)] = (
                        jnp.take(axv, ia) * jnp.take(eav, ie))
                    vbuf[p][e, pl.ds(80, 16)] = jnp.where(
                        lanes < 4, jnp.take(axv, it), zvec)
                return cc

            lax.fori_loop(0, ch // 4, group, 0)
            pltpu.async_copy(vbuf[p], tabv.at[idxd[p]], sc[p], add=True)

        issue(0, 0)

        def pair(cj, carry):
            for p in (0, 1):
                ci = 2 * cj + p

                @pl.when(ci + 1 < nch)
                def _():
                    issue(ci + 1, p ^ 1)

                @pl.when(ci < nch)
                def _():
                    compute(ci, p)

            return carry

        lax.fori_loop(0, (nch + 1) // 2, pair, 0)
        wait_sc(0)
        wait_sc(1)
        plsc.subcore_barrier()

        def dump(j, cc):
            blk = sid + _NS * j

            @pl.when(blk < nfull)
            def _():
                @pl.when(cid == 0)
                def _():
                    pltpu.sync_copy(tabv.at[pl.ds(blk * 128, 128)],
                                    outa.at[pl.ds(blk * 128, 128)])

                @pl.when(cid == 1)
                def _():
                    pltpu.sync_copy(tabv.at[pl.ds(blk * 128, 128)],
                                    outb.at[pl.ds(blk * 128, 128)])

            return cc

        lax.fori_loop(0, nfull // _NS + 1, dump, 0)
        tail = _N - nfull * 128

        @pl.when(sid == 1)
        def _dtail():
            @pl.when(cid == 0)
            def _():
                pltpu.sync_copy(tabv.at[pl.ds(nfull * 128, tail)],
                                outa.at[pl.ds(nfull * 128, tail)])

            @pl.when(cid == 1)
            def _():
                pltpu.sync_copy(tabv.at[pl.ds(nfull * 128, tail)],
                                outb.at[pl.ds(nfull * 128, tail)])

    return pass_b0


def _make_pass_c(ch):
    ept = _E // _NTILES
    nch = ept // ch
    nfull = _N // 128

    @functools.partial(
        pl.kernel,
        out_type=(
            jax.ShapeDtypeStruct((_N, 128), jnp.float32),
            jax.ShapeDtypeStruct((_N, 128), jnp.float32),
        ),
        mesh=_sc_mesh(),
        scratch_types=[
            [pltpu.VMEM((ch,), jnp.int32)] * 2,
            [pltpu.VMEM((ch, 128), jnp.float32)] * 2,
            [pltpu.VMEM((ch * 4,), jnp.float32)] * 2,
            [pltpu.VMEM((ch * 4,), jnp.float32)] * 2,
            pltpu.VMEM((128, 128), jnp.float32),
            pltpu.VMEM_SHARED((_N, 128), jnp.float32),
            [pltpu.SemaphoreType.DMA] * 2,
            [pltpu.SemaphoreType.DMA] * 2,
        ],
    )
    def pass_c(dsti, aexp, ea, outs0, outs1,
               idxd, statbuf, axbuf, eabuf, zb, tabs, sg, sc):
        cid = lax.axis_index("c")
        sid = lax.axis_index("s")
        zv = jnp.zeros((16,), jnp.float32)

        def z1(r, cc):
            for b in range(8):
                zb[r, pl.ds(b * 16, 16)] = zv
            return cc

        lax.fori_loop(0, 128, z1, 0)

        def z3(r, cc):
            for b in range(8):
                statbuf[0][r, pl.ds(b * 16, 16)] = zv
                statbuf[1][r, pl.ds(b * 16, 16)] = zv
            return cc

        lax.fori_loop(0, ch, z3, 0)

        def zc(j, cc):
            blk = sid + _NS * j

            @pl.when(blk < nfull)
            def _():
                pltpu.sync_copy(zb, tabs.at[pl.ds(blk * 128, 128)])

            return cc

        lax.fori_loop(0, nfull // _NS + 1, zc, 0)

        @pl.when(sid == 0)
        def _ztail():
            pltpu.sync_copy(zb.at[pl.ds(0, _N - nfull * 128)],
                            tabs.at[pl.ds(nfull * 128, _N - nfull * 128)])

        plsc.subcore_barrier()

        lanes = lax.broadcasted_iota(jnp.int32, (16,), 0)
        zvec = jnp.zeros((16,), jnp.float32)

        def wait_sc(p):
            pltpu.make_async_copy(statbuf[p], tabs.at[idxd[p]], sc[p]).wait()

        def issue(ci, p):
            @pl.when(ci >= 2)
            def _():
                wait_sc(p)

            base = (cid * _NS + sid) * ept + ci * ch
            pltpu.sync_copy(dsti.at[pl.ds(base, ch)], idxd[p])
            pltpu.async_copy(aexp.at[pl.ds(base * 4, ch * 4)], axbuf[p],
                             sg[p])
            pltpu.async_copy(ea.at[pl.ds(base * 4, ch * 4)], eabuf[p],
                             sg[p])

        def compute(ci, p):
            pltpu.make_async_copy(
                aexp.at[pl.ds(0, ch * 4)], axbuf[p], sg[p]).wait()
            pltpu.make_async_copy(
                ea.at[pl.ds(0, ch * 4)], eabuf[p], sg[p]).wait()

            def group(g, cc):
                axv = axbuf[p][pl.ds(g * 16, 16)]
                eav = eabuf[p][pl.ds(g * 16, 16)]
                for u in range(4):
                    e = g * 4 + u
                    ia = (lanes >> 2) + 4 * u
                    ie = (lanes & 3) + 4 * u
                    it = jnp.minimum(lanes, 3) + 4 * u
                    statbuf[p][e, pl.ds(0, 16)] = (
                        jnp.take(axv, ia) * jnp.take(eav, ie))
                    statbuf[p][e, pl.ds(16, 16)] = jnp.where(
                        lanes < 4, jnp.take(axv, it), zvec)
                return cc

            lax.fori_loop(0, ch // 4, group, 0)
            pltpu.async_copy(statbuf[p], tabs.at[idxd[p]], sc[p], add=True)

        issue(0, 0)

        def pair(cj, carry):
            for p in (0, 1):
                ci = 2 * cj + p

                @pl.when(ci + 1 < nch)
                def _():
                    issue(ci + 1, p ^ 1)

                @pl.when(ci < nch)
                def _():
                    compute(ci, p)

            return carry

        lax.fori_loop(0, (nch + 1) // 2, pair, 0)
        wait_sc(0)
        wait_sc(1)
        plsc.subcore_barrier()

        def dump(j, cc):
            blk = sid + _NS * j

            @pl.when(blk < nfull)
            def _():
                @pl.when(cid == 0)
                def _():
                    pltpu.sync_copy(tabs.at[pl.ds(blk * 128, 128)],
                                    outs0.at[pl.ds(blk * 128, 128)])

                @pl.when(cid == 1)
                def _():
                    pltpu.sync_copy(tabs.at[pl.ds(blk * 128, 128)],
                                    outs1.at[pl.ds(blk * 128, 128)])

            return cc

        lax.fori_loop(0, nfull // _NS + 1, dump, 0)
        tail = _N - nfull * 128

        @pl.when(sid == 1)
        def _dtail():
            @pl.when(cid == 0)
            def _():
                pltpu.sync_copy(tabs.at[pl.ds(nfull * 128, tail)],
                                outs0.at[pl.ds(nfull * 128, tail)])

            @pl.when(cid == 1)
            def _():
                pltpu.sync_copy(tabs.at[pl.ds(nfull * 128, tail)],
                                outs1.at[pl.ds(nfull * 128, tail)])

    return pass_c


def _make_combine(dout):
    c = dout // _HEADS
    f = dout // 2
    bm = 1000

    def body(a_ref, b_ref, st0_ref, st1_ref, s_ref, wet_ref, o_ref):
        st = st0_ref[...] + st1_ref[...]
        parts = []
        for h in range(_HEADS):
            half = a_ref if h < _HEADS // 2 else b_ref
            lo = (h * c) % f
            agg = half[:, lo:lo + c]
            tea = st[:, 4 * h:4 * h + 1] * wet_ref[0:1, h * c:(h + 1) * c]
            for d in range(1, 4):
                tea = tea + (st[:, 4 * h + d:4 * h + d + 1]
                             * wet_ref[d:d + 1, h * c:(h + 1) * c])
            asum = st[:, 16 + h:17 + h]
            parts.append((agg + tea) / (asum + 1e-16)
                         + s_ref[:, h * c:(h + 1) * c])
        o_ref[...] = jnp.concatenate(parts, axis=1)

    wv = _pad128(f)

    def run(outa, outb, st0, st1, s, We_pad):
        return pl.pallas_call(
            body,
            grid=(_N // bm,),
            in_specs=[
                pl.BlockSpec((bm, wv), lambda i: (i, 0)),
                pl.BlockSpec((bm, wv), lambda i: (i, 0)),
                pl.BlockSpec((bm, 128), lambda i: (i, 0)),
                pl.BlockSpec((bm, 128), lambda i: (i, 0)),
                pl.BlockSpec((bm, dout), lambda i: (i, 0)),
                pl.BlockSpec((8, dout), lambda i: (0, 0)),
            ],
            out_specs=pl.BlockSpec((bm, dout), lambda i: (i, 0)),
            out_shape=jax.ShapeDtypeStruct((_N, dout), jnp.float32),
        )(outa, outb, st0, st1, s, We_pad)

    return run


def _norm_body(x_ref, w_ref, b_ref, ms_ref, o_ref):
    x = x_ref[...]
    n = x.shape[0]
    mean = jnp.sum(x, axis=0, keepdims=True) * (1.0 / n)
    cent = x - mean * ms_ref[...]
    var = jnp.sum(cent * cent, axis=0, keepdims=True) * (1.0 / n)
    y = w_ref[...] * cent / jnp.sqrt(var + 1e-5) + b_ref[...]
    o_ref[...] = jnp.maximum(y, 0.0)


def _graph_norm_relu(x, w, b, ms):
    n, d = x.shape
    return pl.pallas_call(
        _norm_body,
        out_shape=jax.ShapeDtypeStruct((n, d), jnp.float32),
    )(x, w.reshape(1, d), b.reshape(1, d), ms.reshape(1, d))


def _make_combine0():
    dout, c = 64, 16
    bm = 1000

    def body(a_ref, b_ref, s_ref, wet_ref, o_ref):
        st = a_ref[...] + b_ref[...]
        parts = []
        for h in range(_HEADS):
            agg = st[:, h * c:(h + 1) * c]
            tea = st[:, 64 + 4 * h:64 + 4 * h + 1] * wet_ref[0:1,
                                                            h * c:(h + 1) * c]
            for d in range(1, 4):
                tea = tea + (st[:, 64 + 4 * h + d:64 + 4 * h + d + 1]
                             * wet_ref[d:d + 1, h * c:(h + 1) * c])
            asum = st[:, 80 + h:81 + h]
            parts.append((agg + tea) / (asum + 1e-16)
                         + s_ref[:, h * c:(h + 1) * c])
        o_ref[...] = jnp.concatenate(parts, axis=1)

    def run(outa, outb, s, We_pad):
        return pl.pallas_call(
            body,
            grid=(_N // bm,),
            in_specs=[
                pl.BlockSpec((bm, 128), lambda i: (i, 0)),
                pl.BlockSpec((bm, 128), lambda i: (i, 0)),
                pl.BlockSpec((bm, dout), lambda i: (i, 0)),
                pl.BlockSpec((8, dout), lambda i: (0, 0)),
            ],
            out_specs=pl.BlockSpec((bm, dout), lambda i: (i, 0)),
            out_shape=jax.ShapeDtypeStruct((_N, dout), jnp.float32),
        )(outa, outb, s, We_pad)

    return run


def _gram_body(h_ref, o_ref, acc_ref):
    i = pl.program_id(0)

    @pl.when(i == 0)
    def _init():
        acc_ref[...] = jnp.zeros_like(acc_ref)

    h = h_ref[...]
    acc_ref[...] += lax.dot_general(
        h, h, (((0,), (0,)), ((), ())), preferred_element_type=jnp.float32
    )

    @pl.when(i == pl.num_programs(0) - 1)
    def _fin():
        a = acc_ref[...]
        mn = jnp.min(a)
        mx = jnp.max(a)
        o_ref[...] = (a - mn) / (mx - mn + 1e-8)


def _gram_norm(h, bm=1000):
    n, d = h.shape
    return pl.pallas_call(
        _gram_body,
        grid=(n // bm,),
        in_specs=[pl.BlockSpec((bm, d), lambda i: (i, 0))],
        out_specs=pl.BlockSpec((d, d), lambda i: (0, 0)),
        out_shape=jax.ShapeDtypeStruct((d, d), jnp.float32),
        scratch_shapes=[pltpu.VMEM((d, d), jnp.float32)],
    )(h)


def _layer(x, dsti, srci, ea_flat, Wq, bq, Wk, bk, Wv, bv, We, be, Ws, bs,
           gw, gb, gms):
    din, dout = Wq.shape
    c = dout // _HEADS
    Wcat = jnp.concatenate([Wq, Wk, Wv, Ws], axis=1)
    bcat = jnp.concatenate([bq, bk + be, bv + be, bs], axis=0)
    We_pad = jnp.pad(We, ((0, 4), (0, 0)))
    head_of_j = jnp.arange(dout) // c
    wcols = [jnp.where(head_of_j == h, We[d], 0.0)
             for h in range(_HEADS) for d in range(4)]
    WeT2 = jnp.stack(wcols, axis=1)
    qp, kt, va, vb, s = _make_proj(din, dout)(x, Wcat, bcat, WeT2)
    ch_a = 200 if dout == 64 else 80
    aexp = _make_pass_a(dout, ch_a)(dsti, srci, qp, kt, ea_flat)
    if dout == 64:
        outa, outb = _make_pass_b0(80)(dsti, srci, va, aexp, ea_flat)
        h = _make_combine0()(outa, outb, s, We_pad)
    else:
        outa, outb = _make_pass_b(dout, 80)(dsti, srci, va, vb, aexp)
        st0, st1 = _make_pass_c(80)(dsti, aexp, ea_flat)
        h = _make_combine(dout)(outa, outb, st0, st1, s, We_pad)
    return _graph_norm_relu(h, gw, gb, gms)


def kernel(x, edge_index, edge_attr,
           Wq0, bq0, Wk0, bk0, Wv0, bv0, We0, be0, Ws0, bs0, gn_w0, gn_b0, gn_ms0,
           Wq1, bq1, Wk1, bk1, Wv1, bv1, We1, be1, Ws1, bs1, gn_w1, gn_b1, gn_ms1):
    srci = edge_index[0]
    dsti = edge_index[1]
    ea_flat = edge_attr.reshape(_E * 4)
    h = _layer(x, dsti, srci, ea_flat, Wq0, bq0, Wk0, bk0, Wv0, bv0, We0, be0,
               Ws0, bs0, gn_w0, gn_b0, gn_ms0)
    h = _layer(h, dsti, srci, ea_flat, Wq1, bq1, Wk1, bk1, Wv1, bv1, We1, be1,
               Ws1, bs1, gn_w1, gn_b1, gn_ms1)
    xt = _gram_norm(h)
    iu = np.triu_indices(_NTGT, k=1)
    return xt[iu[0], iu[1]].reshape(-1, 1)


# speedup vs baseline: 1.3066x; 1.3066x over previous
import functools
import numpy as np
import jax
import jax.numpy as jnp
from jax import lax
from jax.experimental import pallas as pl
from jax.experimental.pallas import tpu as pltpu
from jax.experimental.pallas import tpu_sc as plsc

_N = 10000
_E = 320000
_HEADS = 4
_NTGT = 256

_NC = 2
_NS = 16
_NTILES = _NC * _NS
_CH = 80


def _pad128(w):
    return (w + 127) // 128 * 128


def _sc_mesh():
    return plsc.VectorSubcoreMesh(
        core_axis_name="c", subcore_axis_name="s",
        num_cores=_NC, num_subcores=_NS)


def _make_proj(din, dout):
    c = dout // _HEADS
    f = dout // 2
    wq = _pad128(dout + 16)
    wk = _pad128(dout)
    wv = _pad128(f)
    bm = 1000
    kcat = 4 * dout

    def body(x_ref, wcat_ref, bcat_ref, wet_ref, o_qp, o_k, o_va, o_vb, o_s):
        p = (jnp.dot(x_ref[...], wcat_ref[...],
                     preferred_element_type=jnp.float32) + bcat_ref[...])
        q = p[:, :dout]
        pt = jnp.dot(q, wet_ref[...], preferred_element_type=jnp.float32)
        pcols = [q, pt]
        if wq > dout + 16:
            pcols.append(jnp.zeros((bm, wq - dout - 16), jnp.float32))
        o_qp[...] = jnp.concatenate(pcols, axis=1)
        kcols = [p[:, dout:2 * dout]]
        if wk > dout:
            kcols.append(jnp.zeros((bm, wk - dout), jnp.float32))
        o_k[...] = jnp.concatenate(kcols, axis=1) if len(kcols) > 1 else kcols[0]
        if dout == 64:
            o_va[...] = jnp.concatenate(
                [p[:, 2 * dout:3 * dout],
                 jnp.zeros((bm, wv - dout), jnp.float32)], axis=1)
            o_vb[...] = jnp.zeros((bm, wv), jnp.float32)
        else:
            vpad = [jnp.zeros((bm, wv - f), jnp.float32)] if wv > f else []
            va = [p[:, 2 * dout:2 * dout + f]] + vpad
            vb = [p[:, 2 * dout + f:3 * dout]] + vpad
            o_va[...] = jnp.concatenate(va, axis=1) if len(va) > 1 else va[0]
            o_vb[...] = jnp.concatenate(vb, axis=1) if len(vb) > 1 else vb[0]
        o_s[...] = p[:, 3 * dout:]

    def run(x, Wcat, bcat, WeT2):
        return pl.pallas_call(
            body,
            grid=(_N // bm,),
            in_specs=[
                pl.BlockSpec((bm, din), lambda i: (i, 0)),
                pl.BlockSpec((din, kcat), lambda i: (0, 0)),
                pl.BlockSpec((1, kcat), lambda i: (0, 0)),
                pl.BlockSpec((dout, 16), lambda i: (0, 0)),
            ],
            out_specs=[
                pl.BlockSpec((bm, wq), lambda i: (i, 0)),
                pl.BlockSpec((bm, wk), lambda i: (i, 0)),
                pl.BlockSpec((bm, wv), lambda i: (i, 0)),
                pl.BlockSpec((bm, wv), lambda i: (i, 0)),
                pl.BlockSpec((bm, dout), lambda i: (i, 0)),
            ],
            out_shape=[
                jax.ShapeDtypeStruct((_N, wq), jnp.float32),
                jax.ShapeDtypeStruct((_N, wk), jnp.float32),
                jax.ShapeDtypeStruct((_N, wv), jnp.float32),
                jax.ShapeDtypeStruct((_N, wv), jnp.float32),
                jax.ShapeDtypeStruct((_N, dout), jnp.float32),
            ],
        )(x, Wcat, bcat.reshape(1, kcat), WeT2)

    return run


def _make_pass_a(dout, ch):
    c = dout // _HEADS
    nb = c // 16
    ept = _E // _NTILES
    nch = ept // ch
    wq = _pad128(dout + 16)
    wk = _pad128(dout)
    inv = float(1.0 / np.sqrt(c))

    @functools.partial(
        pl.kernel,
        out_type=jax.ShapeDtypeStruct((_E * _HEADS,), jnp.float32),
        mesh=_sc_mesh(),
        scratch_types=[
            [pltpu.VMEM((ch,), jnp.int32)] * 2,
            [pltpu.VMEM((ch,), jnp.int32)] * 2,
            [pltpu.VMEM((ch, wq), jnp.float32)] * 2,
            [pltpu.VMEM((ch, wk), jnp.float32)] * 2,
            [pltpu.VMEM((ch * 4,), jnp.float32)] * 2,
            [pltpu.VMEM((ch * 4,), jnp.float32)] * 2,
            [pltpu.SemaphoreType.DMA] * 2,
            [pltpu.SemaphoreType.DMA] * 2,
            [pltpu.SemaphoreType.DMA] * 2,
        ],
    )
    def pass_a(dsti, srci, qp, kt, ea, aexp_out,
               idxd, idxs, qb, kb, eb, ab, sg, so, si):
        wid = lax.axis_index("s") * _NC + lax.axis_index("c")
        base0 = wid * ept
        lanes = lax.broadcasted_iota(jnp.int32, (16,), 0)
        perms = [lanes ^ kk for kk in (8, 4, 2, 1)]

        def prefetch(ci, p):
            base = base0 + ci * ch
            pltpu.async_copy(dsti.at[pl.ds(base, ch)], idxd[p], si[p])
            pltpu.async_copy(srci.at[pl.ds(base, ch)], idxs[p], si[p])

        def issue(ci, p):
            base = base0 + ci * ch
            pltpu.make_async_copy(dsti.at[pl.ds(0, ch)], idxd[p],
                                  si[p]).wait()
            pltpu.make_async_copy(srci.at[pl.ds(0, ch)], idxs[p],
                                  si[p]).wait()
            pltpu.async_copy(qp.at[idxd[p]], qb[p], sg[p])
            pltpu.async_copy(kt.at[idxs[p]], kb[p], sg[p])
            pltpu.async_copy(ea.at[pl.ds(base * 4, ch * 4)], eb[p], sg[p])

        def wait_in(p):
            pltpu.make_async_copy(qp.at[idxd[p]], qb[p], sg[p]).wait()
            pltpu.make_async_copy(kt.at[idxs[p]], kb[p], sg[p]).wait()
            pltpu.make_async_copy(ea.at[pl.ds(0, ch * 4)], eb[p], sg[p]).wait()

        def compute(ci, p):
            def group(g, cc):
                eav = eb[p][pl.ds(g * 16, 16)]
                vals = jnp.zeros((16,), jnp.float32)
                for u in range(4):
                    e = g * 4 + u
                    pv = qb[p][e, pl.ds(dout, 16)]
                    for h in range(_HEADS):
                        acc = (qb[p][e, pl.ds(h * c, 16)]
                               * kb[p][e, pl.ds(h * c, 16)])
                        for b in range(1, nb):
                            acc = acc + (qb[p][e, pl.ds(h * c + b * 16, 16)]
                                         * kb[p][e, pl.ds(h * c + b * 16, 16)])
                        for pp in perms:
                            acc = acc + jnp.take(acc, pp)
                        t = (eav[4 * u] * pv[4 * h]
                             + eav[4 * u + 1] * pv[4 * h + 1]
                             + eav[4 * u + 2] * pv[4 * h + 2]
                             + eav[4 * u + 3] * pv[4 * h + 3])
                        vals = jnp.where(lanes == (4 * u + h),
                                         (acc + t) * inv, vals)
                ab[p][pl.ds(g * 16, 16)] = jnp.exp(vals)
                return cc

            lax.fori_loop(0, ch // 4, group, 0)
            base = base0 + ci * ch
            pltpu.async_copy(ab[p], aexp_out.at[pl.ds(base * 4, ch * 4)],
                             so[p])

        def wait_out(p):
            pltpu.make_async_copy(ab[p], aexp_out.at[pl.ds(0, ch * 4)],
                                  so[p]).wait()

        prefetch(0, 0)
        issue(0, 0)
        prefetch(1, 1)

        def pair(cj, carry):
            for p in (0, 1):
                ci = 2 * cj + p

                @pl.when(ci + 1 < nch)
                def _():
                    issue(ci + 1, p ^ 1)

                @pl.when(ci < nch)
                def _():
                    wait_in(p)

                    @pl.when(ci + 2 < nch)
                    def _():
                        prefetch(ci + 2, p)

                    @pl.when(ci >= 2)
                    def _():
                        wait_out(p)

                    compute(ci, p)

            return carry

        lax.fori_loop(0, (nch + 1) // 2, pair, 0)
        wait_out(0)
        wait_out(1)

    return pass_a


def _make_pass_b(dout, ch):
    c = dout // _HEADS
    f = dout // 2
    wv = _pad128(f)
    epsc = _E // _NS
    nch = epsc // ch
    nfull = _N // 128

    @functools.partial(
        pl.kernel,
        out_type=(
            jax.ShapeDtypeStruct((_N, wv), jnp.float32),
            jax.ShapeDtypeStruct((_N, wv), jnp.float32),
        ),
        mesh=_sc_mesh(),
        scratch_types=[
            [pltpu.VMEM((ch,), jnp.int32)] * 2,
            [pltpu.VMEM((ch,), jnp.int32)] * 2,
            [pltpu.VMEM((ch, wv), jnp.float32)] * 2,
            [pltpu.VMEM((ch * 4,), jnp.float32)] * 2,
            pltpu.VMEM((128, wv), jnp.float32),
            pltpu.VMEM_SHARED((_N, wv), jnp.float32),
            [pltpu.SemaphoreType.DMA] * 2,
            [pltpu.SemaphoreType.DMA] * 2,
            [pltpu.SemaphoreType.DMA] * 2,
            [pltpu.SemaphoreType.DMA] * 2,
        ],
    )
    def pass_b(dsti, srci, va, vb, aexp, outa, outb,
               idxd, idxs, vbuf, axbuf, zb, tabv, sg, sc, si, sd):
        cid = lax.axis_index("c")
        sid = lax.axis_index("s")
        zv = jnp.zeros((16,), jnp.float32)

        def z1(r, cc):
            for b in range(wv // 16):
                zb[r, pl.ds(b * 16, 16)] = zv
            return cc

        lax.fori_loop(0, 128, z1, 0)

        def zc(j, cc):
            blk = sid + _NS * j

            @pl.when(blk < nfull)
            def _():
                pltpu.sync_copy(zb, tabv.at[pl.ds(blk * 128, 128)])

            return cc

        lax.fori_loop(0, nfull // _NS + 1, zc, 0)

        @pl.when(sid == 0)
        def _ztail():
            pltpu.sync_copy(zb.at[pl.ds(0, _N - nfull * 128)],
                            tabv.at[pl.ds(nfull * 128, _N - nfull * 128)])

        plsc.subcore_barrier()

        def wait_sc(p):
            pltpu.make_async_copy(vbuf[p], tabv.at[idxd[p]], sc[p]).wait()

        def prefetch(ci, p):
            base = sid * epsc + ci * ch
            pltpu.async_copy(srci.at[pl.ds(base, ch)], idxs[p], si[p])

        def issue(ci, p):
            @pl.when(ci >= 2)
            def _():
                wait_sc(p)

            base = sid * epsc + ci * ch
            pltpu.async_copy(dsti.at[pl.ds(base, ch)], idxd[p], sd[p])
            pltpu.make_async_copy(srci.at[pl.ds(0, ch)], idxs[p],
                                  si[p]).wait()
            pltpu.async_copy(aexp.at[pl.ds(base * 4, ch * 4)], axbuf[p],
                             sg[p])

            @pl.when(cid == 0)
            def _():
                pltpu.async_copy(va.at[idxs[p]], vbuf[p], sg[p])

            @pl.when(cid == 1)
            def _():
                pltpu.async_copy(vb.at[idxs[p]], vbuf[p], sg[p])

        def compute(ci, p):
            pltpu.make_async_copy(va.at[idxs[p]], vbuf[p], sg[p]).wait()
            pltpu.make_async_copy(
                aexp.at[pl.ds(0, ch * 4)], axbuf[p], sg[p]).wait()

            @pl.when(ci + 2 < nch)
            def _():
                prefetch(ci + 2, p)

            def scale(coff):
                def group(g, cc):
                    axv = axbuf[p][pl.ds(g * 16, 16)]
                    for u in range(4):
                        e = g * 4 + u
                        for b in range(f // 16):
                            h = (coff + b * 16) // c
                            vbuf[p][e, pl.ds(b * 16, 16)] = (
                                vbuf[p][e, pl.ds(b * 16, 16)]
                                * axv[4 * u + h])
                    return cc

                lax.fori_loop(0, ch // 4, group, 0)

            @pl.when(cid == 0)
            def _():
                scale(0)

            @pl.when(cid == 1)
            def _():
                scale(f)

            pltpu.make_async_copy(dsti.at[pl.ds(0, ch)], idxd[p],
                                  sd[p]).wait()
            pltpu.async_copy(vbuf[p], tabv.at[idxd[p]], sc[p], add=True)

        prefetch(0, 0)
        issue(0, 0)
        prefetch(1, 1)

        def pair(cj, carry):
            for p in (0, 1):
                ci = 2 * cj + p

                @pl.when(ci + 1 < nch)
                def _():
                    issue(ci + 1, p ^ 1)

                @pl.when(ci < nch)
                def _():
                    compute(ci, p)

            return carry

        lax.fori_loop(0, (nch + 1) // 2, pair, 0)
        wait_sc(0)
        wait_sc(1)
        plsc.subcore_barrier()

        def dump(j, cc):
            blk = sid + _NS * j

            @pl.when(blk < nfull)
            def _():
                @pl.when(cid == 0)
                def _():
                    pltpu.sync_copy(tabv.at[pl.ds(blk * 128, 128)],
                                    outa.at[pl.ds(blk * 128, 128)])

                @pl.when(cid == 1)
                def _():
                    pltpu.sync_copy(tabv.at[pl.ds(blk * 128, 128)],
                                    outb.at[pl.ds(blk * 128, 128)])

            return cc

        lax.fori_loop(0, nfull // _NS + 1, dump, 0)
        tail = _N - nfull * 128

        @pl.when(sid == 1)
        def _dtail():
            @pl.when(cid == 0)
            def _():
                pltpu.sync_copy(tabv.at[pl.ds(nfull * 128, tail)],
                                outa.at[pl.ds(nfull * 128, tail)])

            @pl.when(cid == 1)
            def _():
                pltpu.sync_copy(tabv.at[pl.ds(nfull * 128, tail)],
                                outb.at[pl.ds(nfull * 128, tail)])

    return pass_b


def _make_pass_b0(ch):
    dout = 64
    ept = _E // _NTILES
    nch = ept // ch
    nfull = _N // 128

    @functools.partial(
        pl.kernel,
        out_type=(
            jax.ShapeDtypeStruct((_N, 128), jnp.float32),
            jax.ShapeDtypeStruct((_N, 128), jnp.float32),
        ),
        mesh=_sc_mesh(),
        scratch_types=[
            [pltpu.VMEM((ch,), jnp.int32)] * 2,
            [pltpu.VMEM((ch,), jnp.int32)] * 2,
            [pltpu.VMEM((ch, 128), jnp.float32)] * 2,
            [pltpu.VMEM((ch * 4,), jnp.float32)] * 2,
            [pltpu.VMEM((ch * 4,), jnp.float32)] * 2,
            pltpu.VMEM((128, 128), jnp.float32),
            pltpu.VMEM_SHARED((_N, 128), jnp.float32),
            [pltpu.SemaphoreType.DMA] * 2,
            [pltpu.SemaphoreType.DMA] * 2,
            [pltpu.SemaphoreType.DMA] * 2,
            [pltpu.SemaphoreType.DMA] * 2,
        ],
    )
    def pass_b0(dsti, srci, vt, aexp, ea, outa, outb,
                idxd, idxs, vbuf, axbuf, eabuf, zb, tabv, sg, sc, si, sd):
        cid = lax.axis_index("c")
        sid = lax.axis_index("s")
        zv = jnp.zeros((16,), jnp.float32)

        def z1(r, cc):
            for b in range(8):
                zb[r, pl.ds(b * 16, 16)] = zv
            return cc

        lax.fori_loop(0, 128, z1, 0)

        def zc(j, cc):
            blk = sid + _NS * j

            @pl.when(blk < nfull)
            def _():
                pltpu.sync_copy(zb, tabv.at[pl.ds(blk * 128, 128)])

            return cc

        lax.fori_loop(0, nfull // _NS + 1, zc, 0)

        @pl.when(sid == 0)
        def _ztail():
            pltpu.sync_copy(zb.at[pl.ds(0, _N - nfull * 128)],
                            tabv.at[pl.ds(nfull * 128, _N - nfull * 128)])

        plsc.subcore_barrier()

        lanes = lax.broadcasted_iota(jnp.int32, (16,), 0)
        zvec = jnp.zeros((16,), jnp.float32)

        def wait_sc(p):
            pltpu.make_async_copy(vbuf[p], tabv.at[idxd[p]], sc[p]).wait()

        def prefetch(ci, p):
            base = (cid * _NS + sid) * ept + ci * ch
            pltpu.async_copy(srci.at[pl.ds(base, ch)], idxs[p], si[p])

        def issue(ci, p):
            @pl.when(ci >= 2)
            def _():
                wait_sc(p)

            base = (cid * _NS + sid) * ept + ci * ch
            pltpu.async_copy(dsti.at[pl.ds(base, ch)], idxd[p], sd[p])
            pltpu.make_async_copy(srci.at[pl.ds(0, ch)], idxs[p],
                                  si[p]).wait()
            pltpu.async_copy(aexp.at[pl.ds(base * 4, ch * 4)], axbuf[p],
                             sg[p])
            pltpu.async_copy(ea.at[pl.ds(base * 4, ch * 4)], eabuf[p], sg[p])
            pltpu.async_copy(vt.at[idxs[p]], vbuf[p], sg[p])

        def compute(ci, p):
            pltpu.make_async_copy(vt.at[idxs[p]], vbuf[p], sg[p]).wait()
            pltpu.make_async_copy(
                aexp.at[pl.ds(0, ch * 4)], axbuf[p], sg[p]).wait()
            pltpu.make_async_copy(
                ea.at[pl.ds(0, ch * 4)], eabuf[p], sg[p]).wait()

            @pl.when(ci + 2 < nch)
            def _():
                prefetch(ci + 2, p)

            def group(g, cc):
                axv = axbuf[p][pl.ds(g * 16, 16)]
                eav = eabuf[p][pl.ds(g * 16, 16)]
                for u in range(4):
                    e = g * 4 + u
                    for b in range(4):
                        vbuf[p][e, pl.ds(b * 16, 16)] = (
                            vbuf[p][e, pl.ds(b * 16, 16)] * axv[4 * u + b])
                    ia = (lanes >> 2) + 4 * u
                    ie = (lanes & 3) + 4 * u
                    it = jnp.minimum(lanes, 3) + 4 * u
                    vbuf[p][e, pl.ds(64, 16)] = (
                        jnp.take(axv, ia) * jnp.take(eav, ie))
                    vbuf[p][e, pl.ds(80, 16)] = jnp.where(
                        lanes < 4, jnp.take(axv, it), zvec)
                return cc

            lax.fori_loop(0, ch // 4, group, 0)
            pltpu.make_async_copy(dsti.at[pl.ds(0, ch)], idxd[p],
                                  sd[p]).wait()
            pltpu.async_copy(vbuf[p], tabv.at[idxd[p]], sc[p], add=True)

        prefetch(0, 0)
        issue(0, 0)
        prefetch(1, 1)

        def pair(cj, carry):
            for p in (0, 1):
                ci = 2 * cj + p

                @pl.when(ci + 1 < nch)
                def _():
                    issue(ci + 1, p ^ 1)

                @pl.when(ci < nch)
                def _():
                    compute(ci, p)

            return carry

        lax.fori_loop(0, (nch + 1) // 2, pair, 0)
        wait_sc(0)
        wait_sc(1)
        plsc.subcore_barrier()

        def dump(j, cc):
            blk = sid + _NS * j

            @pl.when(blk < nfull)
            def _():
                @pl.when(cid == 0)
                def _():
                    pltpu.sync_copy(tabv.at[pl.ds(blk * 128, 128)],
                                    outa.at[pl.ds(blk * 128, 128)])

                @pl.when(cid == 1)
                def _():
                    pltpu.sync_copy(tabv.at[pl.ds(blk * 128, 128)],
                                    outb.at[pl.ds(blk * 128, 128)])

            return cc

        lax.fori_loop(0, nfull // _NS + 1, dump, 0)
        tail = _N - nfull * 128

        @pl.when(sid == 1)
        def _dtail():
            @pl.when(cid == 0)
            def _():
                pltpu.sync_copy(tabv.at[pl.ds(nfull * 128, tail)],
                                outa.at[pl.ds(nfull * 128, tail)])

            @pl.when(cid == 1)
            def _():
                pltpu.sync_copy(tabv.at[pl.ds(nfull * 128, tail)],
                                outb.at[pl.ds(nfull * 128, tail)])

    return pass_b0


def _make_pass_c(ch):
    ept = _E // _NTILES
    nch = ept // ch
    nfull = _N // 128

    @functools.partial(
        pl.kernel,
        out_type=(
            jax.ShapeDtypeStruct((_N, 128), jnp.float32),
            jax.ShapeDtypeStruct((_N, 128), jnp.float32),
        ),
        mesh=_sc_mesh(),
        scratch_types=[
            [pltpu.VMEM((ch,), jnp.int32)] * 2,
            [pltpu.VMEM((ch, 128), jnp.float32)] * 2,
            [pltpu.VMEM((ch * 4,), jnp.float32)] * 2,
            [pltpu.VMEM((ch * 4,), jnp.float32)] * 2,
            pltpu.VMEM((128, 128), jnp.float32),
            pltpu.VMEM_SHARED((_N, 128), jnp.float32),
            [pltpu.SemaphoreType.DMA] * 2,
            [pltpu.SemaphoreType.DMA] * 2,
            [pltpu.SemaphoreType.DMA] * 2,
        ],
    )
    def pass_c(dsti, aexp, ea, outs0, outs1,
               idxd, statbuf, axbuf, eabuf, zb, tabs, sg, sc, sd):
        cid = lax.axis_index("c")
        sid = lax.axis_index("s")
        zv = jnp.zeros((16,), jnp.float32)

        def z1(r, cc):
            for b in range(8):
                zb[r, pl.ds(b * 16, 16)] = zv
            return cc

        lax.fori_loop(0, 128, z1, 0)

        def z3(r, cc):
            for b in range(8):
                statbuf[0][r, pl.ds(b * 16, 16)] = zv
                statbuf[1][r, pl.ds(b * 16, 16)] = zv
            return cc

        lax.fori_loop(0, ch, z3, 0)

        def zc(j, cc):
            blk = sid + _NS * j

            @pl.when(blk < nfull)
            def _():
                pltpu.sync_copy(zb, tabs.at[pl.ds(blk * 128, 128)])

            return cc

        lax.fori_loop(0, nfull // _NS + 1, zc, 0)

        @pl.when(sid == 0)
        def _ztail():
            pltpu.sync_copy(zb.at[pl.ds(0, _N - nfull * 128)],
                            tabs.at[pl.ds(nfull * 128, _N - nfull * 128)])

        plsc.subcore_barrier()

        lanes = lax.broadcasted_iota(jnp.int32, (16,), 0)
        zvec = jnp.zeros((16,), jnp.float32)

        def wait_sc(p):
            pltpu.make_async_copy(statbuf[p], tabs.at[idxd[p]], sc[p]).wait()

        def issue(ci, p):
            @pl.when(ci >= 2)
            def _():
                wait_sc(p)

            base = (cid * _NS + sid) * ept + ci * ch
            pltpu.async_copy(dsti.at[pl.ds(base, ch)], idxd[p], sd[p])
            pltpu.async_copy(aexp.at[pl.ds(base * 4, ch * 4)], axbuf[p],
                             sg[p])
            pltpu.async_copy(ea.at[pl.ds(base * 4, ch * 4)], eabuf[p],
                             sg[p])

        def compute(ci, p):
            pltpu.make_async_copy(
                aexp.at[pl.ds(0, ch * 4)], axbuf[p], sg[p]).wait()
            pltpu.make_async_copy(
                ea.at[pl.ds(0, ch * 4)], eabuf[p], sg[p]).wait()

            def group(g, cc):
                axv = axbuf[p][pl.ds(g * 16, 16)]
                eav = eabuf[p][pl.ds(g * 16, 16)]
                for u in range(4):
                    e = g * 4 + u
                    ia = (lanes >> 2) + 4 * u
                    ie = (lanes & 3) + 4 * u
                    it = jnp.minimum(lanes, 3) + 4 * u
                    statbuf[p][e, pl.ds(0, 16)] = (
                        jnp.take(axv, ia) * jnp.take(eav, ie))
                    statbuf[p][e, pl.ds(16, 16)] = jnp.where(
                        lanes < 4, jnp.take(axv, it), zvec)
                return cc

            lax.fori_loop(0, ch // 4, group, 0)
            pltpu.make_async_copy(dsti.at[pl.ds(0, ch)], idxd[p],
                                  sd[p]).wait()
            pltpu.async_copy(statbuf[p], tabs.at[idxd[p]], sc[p], add=True)

        issue(0, 0)

        def pair(cj, carry):
            for p in (0, 1):
                ci = 2 * cj + p

                @pl.when(ci + 1 < nch)
                def _():
                    issue(ci + 1, p ^ 1)

                @pl.when(ci < nch)
                def _():
                    compute(ci, p)

            return carry

        lax.fori_loop(0, (nch + 1) // 2, pair, 0)
        wait_sc(0)
        wait_sc(1)
        plsc.subcore_barrier()

        def dump(j, cc):
            blk = sid + _NS * j

            @pl.when(blk < nfull)
            def _():
                @pl.when(cid == 0)
                def _():
                    pltpu.sync_copy(tabs.at[pl.ds(blk * 128, 128)],
                                    outs0.at[pl.ds(blk * 128, 128)])

                @pl.when(cid == 1)
                def _():
                    pltpu.sync_copy(tabs.at[pl.ds(blk * 128, 128)],
                                    outs1.at[pl.ds(blk * 128, 128)])

            return cc

        lax.fori_loop(0, nfull // _NS + 1, dump, 0)
        tail = _N - nfull * 128

        @pl.when(sid == 1)
        def _dtail():
            @pl.when(cid == 0)
            def _():
                pltpu.sync_copy(tabs.at[pl.ds(nfull * 128, tail)],
                                outs0.at[pl.ds(nfull * 128, tail)])

            @pl.when(cid == 1)
            def _():
                pltpu.sync_copy(tabs.at[pl.ds(nfull * 128, tail)],
                                outs1.at[pl.ds(nfull * 128, tail)])

    return pass_c


def _make_combine(dout):
    c = dout // _HEADS
    f = dout // 2
    bm = 1000

    def body(a_ref, b_ref, st0_ref, st1_ref, s_ref, wet_ref, o_ref):
        st = st0_ref[...] + st1_ref[...]
        parts = []
        for h in range(_HEADS):
            half = a_ref if h < _HEADS // 2 else b_ref
            lo = (h * c) % f
            agg = half[:, lo:lo + c]
            tea = st[:, 4 * h:4 * h + 1] * wet_ref[0:1, h * c:(h + 1) * c]
            for d in range(1, 4):
                tea = tea + (st[:, 4 * h + d:4 * h + d + 1]
                             * wet_ref[d:d + 1, h * c:(h + 1) * c])
            asum = st[:, 16 + h:17 + h]
            parts.append((agg + tea) / (asum + 1e-16)
                         + s_ref[:, h * c:(h + 1) * c])
        o_ref[...] = jnp.concatenate(parts, axis=1)

    wv = _pad128(f)

    def run(outa, outb, st0, st1, s, We_pad):
        return pl.pallas_call(
            body,
            grid=(_N // bm,),
            in_specs=[
                pl.BlockSpec((bm, wv), lambda i: (i, 0)),
                pl.BlockSpec((bm, wv), lambda i: (i, 0)),
                pl.BlockSpec((bm, 128), lambda i: (i, 0)),
                pl.BlockSpec((bm, 128), lambda i: (i, 0)),
                pl.BlockSpec((bm, dout), lambda i: (i, 0)),
                pl.BlockSpec((8, dout), lambda i: (0, 0)),
            ],
            out_specs=pl.BlockSpec((bm, dout), lambda i: (i, 0)),
            out_shape=jax.ShapeDtypeStruct((_N, dout), jnp.float32),
        )(outa, outb, st0, st1, s, We_pad)

    return run


def _norm_body(x_ref, w_ref, b_ref, ms_ref, o_ref):
    x = x_ref[...]
    n = x.shape[0]
    mean = jnp.sum(x, axis=0, keepdims=True) * (1.0 / n)
    cent = x - mean * ms_ref[...]
    var = jnp.sum(cent * cent, axis=0, keepdims=True) * (1.0 / n)
    y = w_ref[...] * cent / jnp.sqrt(var + 1e-5) + b_ref[...]
    o_ref[...] = jnp.maximum(y, 0.0)


def _graph_norm_relu(x, w, b, ms):
    n, d = x.shape
    return pl.pallas_call(
        _norm_body,
        out_shape=jax.ShapeDtypeStruct((n, d), jnp.float32),
    )(x, w.reshape(1, d), b.reshape(1, d), ms.reshape(1, d))


def _make_combine0():
    dout, c = 64, 16
    bm = 1000

    def body(a_ref, b_ref, s_ref, wet_ref, o_ref):
        st = a_ref[...] + b_ref[...]
        parts = []
        for h in range(_HEADS):
            agg = st[:, h * c:(h + 1) * c]
            tea = st[:, 64 + 4 * h:64 + 4 * h + 1] * wet_ref[0:1,
                                                            h * c:(h + 1) * c]
            for d in range(1, 4):
                tea = tea + (st[:, 64 + 4 * h + d:64 + 4 * h + d + 1]
                             * wet_ref[d:d + 1, h * c:(h + 1) * c])
            asum = st[:, 80 + h:81 + h]
            parts.append((agg + tea) / (asum + 1e-16)
                         + s_ref[:, h * c:(h + 1) * c])
        o_ref[...] = jnp.concatenate(parts, axis=1)

    def run(outa, outb, s, We_pad):
        return pl.pallas_call(
            body,
            grid=(_N // bm,),
            in_specs=[
                pl.BlockSpec((bm, 128), lambda i: (i, 0)),
                pl.BlockSpec((bm, 128), lambda i: (i, 0)),
                pl.BlockSpec((bm, dout), lambda i: (i, 0)),
                pl.BlockSpec((8, dout), lambda i: (0, 0)),
            ],
            out_specs=pl.BlockSpec((bm, dout), lambda i: (i, 0)),
            out_shape=jax.ShapeDtypeStruct((_N, dout), jnp.float32),
        )(outa, outb, s, We_pad)

    return run


def _gram_body(h_ref, o_ref, acc_ref):
    i = pl.program_id(0)

    @pl.when(i == 0)
    def _init():
        acc_ref[...] = jnp.zeros_like(acc_ref)

    h = h_ref[...]
    acc_ref[...] += lax.dot_general(
        h, h, (((0,), (0,)), ((), ())), preferred_element_type=jnp.float32
    )

    @pl.when(i == pl.num_programs(0) - 1)
    def _fin():
        a = acc_ref[...]
        mn = jnp.min(a)
        mx = jnp.max(a)
        o_ref[...] = (a - mn) / (mx - mn + 1e-8)


def _gram_norm(h, bm=1000):
    n, d = h.shape
    return pl.pallas_call(
        _gram_body,
        grid=(n // bm,),
        in_specs=[pl.BlockSpec((bm, d), lambda i: (i, 0))],
        out_specs=pl.BlockSpec((d, d), lambda i: (0, 0)),
        out_shape=jax.ShapeDtypeStruct((d, d), jnp.float32),
        scratch_shapes=[pltpu.VMEM((d, d), jnp.float32)],
    )(h)


def _layer(x, dsti, srci, ea_flat, Wq, bq, Wk, bk, Wv, bv, We, be, Ws, bs,
           gw, gb, gms):
    din, dout = Wq.shape
    c = dout // _HEADS
    Wcat = jnp.concatenate([Wq, Wk, Wv, Ws], axis=1)
    bcat = jnp.concatenate([bq, bk + be, bv + be, bs], axis=0)
    We_pad = jnp.pad(We, ((0, 4), (0, 0)))
    head_of_j = jnp.arange(dout) // c
    wcols = [jnp.where(head_of_j == h, We[d], 0.0)
             for h in range(_HEADS) for d in range(4)]
    WeT2 = jnp.stack(wcols, axis=1)
    qp, kt, va, vb, s = _make_proj(din, dout)(x, Wcat, bcat, WeT2)
    ch_a = 200 if dout == 64 else 80
    aexp = _make_pass_a(dout, ch_a)(dsti, srci, qp, kt, ea_flat)
    if dout == 64:
        outa, outb = _make_pass_b0(80)(dsti, srci, va, aexp, ea_flat)
        h = _make_combine0()(outa, outb, s, We_pad)
    else:
        outa, outb = _make_pass_b(dout, 80)(dsti, srci, va, vb, aexp)
        st0, st1 = _make_pass_c(80)(dsti, aexp, ea_flat)
        h = _make_combine(dout)(outa, outb, st0, st1, s, We_pad)
    return _graph_norm_relu(h, gw, gb, gms)


def kernel(x, edge_index, edge_attr,
           Wq0, bq0, Wk0, bk0, Wv0, bv0, We0, be0, Ws0, bs0, gn_w0, gn_b0, gn_ms0,
           Wq1, bq1, Wk1, bk1, Wv1, bv1, We1, be1, Ws1, bs1, gn_w1, gn_b1, gn_ms1):
    srci = edge_index[0]
    dsti = edge_index[1]
    ea_flat = edge_attr.reshape(_E * 4)
    h = _layer(x, dsti, srci, ea_flat, Wq0, bq0, Wk0, bk0, Wv0, bv0, We0, be0,
               Ws0, bs0, gn_w0, gn_b0, gn_ms0)
    h = _layer(h, dsti, srci, ea_flat, Wq1, bq1, Wk1, bk1, Wv1, bv1, We1, be1,
               Ws1, bs1, gn_w1, gn_b1, gn_ms1)
    xt = _gram_norm(h)
    iu = np.triu_indices(_NTGT, k=1)
    return xt[iu[0], iu[1]].reshape(-1, 1)
